# baseline (speedup 1.0000x reference)
import functools
import math

import jax
import jax.numpy as jnp
from jax import lax
from jax.experimental import pallas as pl
from jax.experimental.pallas import tpu as pltpu
from jax.experimental.pallas import tpu_sc as plsc

_T = 5
_GAMMA = 0.015
_EPS = 0.0001

_B = 16
_NA = 2048
_NCJ = 1024
_PER = _NA + _NCJ
_NATOT = _B * _NA
_NCTOT = _B * _NCJ

_NCORES = 2
_NSUB = 16
_NW = _NCORES * _NSUB


def _mesh():
    return plsc.VectorSubcoreMesh(core_axis_name="c", subcore_axis_name="s")


def _zero_vmem(ref, n):
    @plsc.parallel_loop(0, n // 16, unroll=8)
    def _(i):
        ref[pl.ds(i * 16, 16)] = jnp.zeros((16,), jnp.float32)


def _bat16(n):
    return lax.shift_right_logical(lax.shift_right_logical(n, 10) * 21846, 16)


def _pass_a2c_first(la, eflat):
    e = eflat.shape[1] // 2
    ept = e // _NW

    @functools.partial(
        pl.kernel,
        out_type=[jax.ShapeDtypeStruct((_NW, _NCTOT), jnp.float32),
                  jax.ShapeDtypeStruct((e,), jnp.int32)],
        mesh=_mesh(),
        compiler_params=pltpu.CompilerParams(needs_layout_passes=False),
        scratch_types=[
            pltpu.VMEM((_NATOT,), jnp.float32),
            pltpu.VMEM((_NCTOT,), jnp.float32),
            pltpu.VMEM((ept,), jnp.int32),
            pltpu.VMEM((ept,), jnp.int32),
            pltpu.SemaphoreType.DMA,
            pltpu.SemaphoreType.DMA,
            pltpu.SemaphoreType.DMA,
        ],
    )
    def k(la_h, ef_h, out_h, pk_h, tbl, acc, sbuf, dbuf, sem, sem2, sem3):
        wid = lax.axis_index("s") * _NCORES + lax.axis_index("c")
        base = wid * ept
        d1 = pltpu.async_copy(la_h, tbl, sem)
        d2 = pltpu.async_copy(ef_h.at[0, pl.ds(base, ept)], sbuf, sem2)
        d3 = pltpu.async_copy(ef_h.at[1, pl.ds(base, ept)], dbuf, sem3)
        _zero_vmem(acc, _NCTOT)
        d2.wait()
        d3.wait()

        @plsc.parallel_loop(0, ept // 16, unroll=8)
        def _(j):
            srca = sbuf[pl.ds(j * 16, 16)]
            dstc = dbuf[pl.ds(j * 16, 16)]
            sa = srca - _bat16(srca) * _NCJ
            dc = dstc - _bat16(dstc) * _NA - _NA
            sbuf[pl.ds(j * 16, 16)] = sa | lax.shift_left(dc, 15)

        dout = pltpu.async_copy(sbuf, pk_h.at[pl.ds(base, ept)], sem2)
        d1.wait()

        @plsc.parallel_loop(0, ept // 16, unroll=16)
        def _(j):
            p = sbuf[pl.ds(j * 16, 16)]
            si = p & 0x7FFF
            di = lax.shift_right_logical(p, 15)
            v = plsc.load_gather(tbl, [si])
            plsc.addupdate_scatter(acc, [di], v)

        pltpu.sync_copy(acc, out_h.at[wid])
        dout.wait()

    return k(la, eflat)


def _pass_c2a_first(xc, eflat, eci, wtbl):
    e = eflat.shape[1] // 2
    ept = e // _NW
    ncl = wtbl.shape[0]

    @functools.partial(
        pl.kernel,
        out_type=[jax.ShapeDtypeStruct((_NW, _NATOT), jnp.float32),
                  jax.ShapeDtypeStruct((e,), jnp.int32),
                  jax.ShapeDtypeStruct((e,), jnp.int32)],
        mesh=_mesh(),
        compiler_params=pltpu.CompilerParams(needs_layout_passes=False),
        scratch_types=[
            pltpu.VMEM((_NCTOT,), jnp.float32),
            pltpu.VMEM((ncl,), jnp.float32),
            pltpu.VMEM((_NATOT,), jnp.float32),
            pltpu.VMEM((ept,), jnp.int32),
            pltpu.VMEM((ept,), jnp.int32),
            pltpu.VMEM((ept,), jnp.int32),
            pltpu.SemaphoreType.DMA,
            pltpu.SemaphoreType.DMA,
            pltpu.SemaphoreType.DMA,
            pltpu.SemaphoreType.DMA,
            pltpu.SemaphoreType.DMA,
        ],
    )
    def k(xc_h, ef_h, eci_h, w_h, out_h, pk_h, da_h,
          tbl, wv, acc, sbuf, dbuf, ebuf, sem, sem2, sem3, sem4, sem5):
        wid = lax.axis_index("s") * _NCORES + lax.axis_index("c")
        base = wid * ept
        d1 = pltpu.async_copy(xc_h, tbl, sem)
        d2 = pltpu.async_copy(w_h, wv, sem2)
        d3 = pltpu.async_copy(ef_h.at[0, pl.ds(e + base, ept)], sbuf, sem3)
        d4 = pltpu.async_copy(ef_h.at[1, pl.ds(e + base, ept)], dbuf, sem4)
        d5 = pltpu.async_copy(eci_h.at[pl.ds(e + base, ept)], ebuf, sem5)
        _zero_vmem(acc, _NATOT)
        d3.wait()
        d4.wait()
        d5.wait()

        @plsc.parallel_loop(0, ept // 16, unroll=8)
        def _(j):
            srcc = sbuf[pl.ds(j * 16, 16)]
            dsta = dbuf[pl.ds(j * 16, 16)]
            ec = ebuf[pl.ds(j * 16, 16)]
            sc = srcc - _bat16(srcc) * _NA - _NA
            sbuf[pl.ds(j * 16, 16)] = sc | lax.shift_left(ec, 14)
            dbuf[pl.ds(j * 16, 16)] = dsta - _bat16(dsta) * _NCJ

        do1 = pltpu.async_copy(sbuf, pk_h.at[pl.ds(base, ept)], sem3)
        do2 = pltpu.async_copy(dbuf, da_h.at[pl.ds(base, ept)], sem4)
        d1.wait()
        d2.wait()

        @plsc.parallel_loop(0, ept // 16, unroll=16)
        def _(j):
            p = sbuf[pl.ds(j * 16, 16)]
            di = dbuf[pl.ds(j * 16, 16)]
            si = p & 0x3FFF
            ki = lax.shift_right_logical(p, 14)
            v = plsc.load_gather(tbl, [si])
            w = plsc.load_gather(wv, [ki])
            plsc.addupdate_scatter(acc, [di], jnp.exp(v * w))

        pltpu.sync_copy(acc, out_h.at[wid])
        do1.wait()
        do2.wait()

    return k(xc, eflat, eci, wtbl)


def _pass_a2c(la, pk):
    e = pk.shape[0]
    ept = e // _NW

    @functools.partial(
        pl.kernel,
        out_type=jax.ShapeDtypeStruct((_NW, _NCTOT), jnp.float32),
        mesh=_mesh(),
        compiler_params=pltpu.CompilerParams(needs_layout_passes=False),
        scratch_types=[
            pltpu.VMEM((_NATOT,), jnp.float32),
            pltpu.VMEM((_NCTOT,), jnp.float32),
            pltpu.VMEM((ept,), jnp.int32),
            pltpu.SemaphoreType.DMA,
        ],
    )
    def k(la_h, pk_h, out_h, tbl, acc, pbuf, sem):
        wid = lax.axis_index("s") * _NCORES + lax.axis_index("c")
        base = wid * ept
        d1 = pltpu.async_copy(la_h, tbl, sem)
        d2 = pltpu.async_copy(pk_h.at[pl.ds(base, ept)], pbuf, sem)
        _zero_vmem(acc, _NCTOT)
        d1.wait()
        d2.wait()

        @plsc.parallel_loop(0, ept // 16, unroll=16)
        def _(j):
            p = pbuf[pl.ds(j * 16, 16)]
            si = p & 0x7FFF
            di = lax.shift_right_logical(p, 15)
            v = plsc.load_gather(tbl, [si])
            plsc.addupdate_scatter(acc, [di], v)

        pltpu.sync_copy(acc, out_h.at[wid])

    return k(la, pk)


def _pass_c2a(xc, pk, dst, wtbl):
    e = pk.shape[0]
    ept = e // _NW
    ncl = wtbl.shape[0]

    @functools.partial(
        pl.kernel,
        out_type=jax.ShapeDtypeStruct((_NW, _NATOT), jnp.float32),
        mesh=_mesh(),
        compiler_params=pltpu.CompilerParams(needs_layout_passes=False),
        scratch_types=[
            pltpu.VMEM((_NCTOT,), jnp.float32),
            pltpu.VMEM((ncl,), jnp.float32),
            pltpu.VMEM((_NATOT,), jnp.float32),
            pltpu.VMEM((ept,), jnp.int32),
            pltpu.VMEM((ept,), jnp.int32),
            pltpu.SemaphoreType.DMA,
        ],
    )
    def k(xc_h, pk_h, dst_h, w_h, out_h, tbl, wv, acc, pbuf, dbuf, sem):
        wid = lax.axis_index("s") * _NCORES + lax.axis_index("c")
        base = wid * ept
        d1 = pltpu.async_copy(xc_h, tbl, sem)
        d2 = pltpu.async_copy(w_h, wv, sem)
        d3 = pltpu.async_copy(pk_h.at[pl.ds(base, ept)], pbuf, sem)
        d4 = pltpu.async_copy(dst_h.at[pl.ds(base, ept)], dbuf, sem)
        _zero_vmem(acc, _NATOT)
        d1.wait()
        d2.wait()
        d3.wait()
        d4.wait()

        @plsc.parallel_loop(0, ept // 16, unroll=16)
        def _(j):
            p = pbuf[pl.ds(j * 16, 16)]
            di = dbuf[pl.ds(j * 16, 16)]
            si = p & 0x3FFF
            ki = lax.shift_right_logical(p, 14)
            v = plsc.load_gather(tbl, [si])
            w = plsc.load_gather(wv, [ki])
            plsc.addupdate_scatter(acc, [di], jnp.exp(v * w))

        pltpu.sync_copy(acc, out_h.at[wid])

    return k(xc, pk, dst, wtbl)


def _tc_combine_a(partials2, xc1):
    n = xc1.shape[0]
    g = 4
    blk = n // g

    def body(pr, xr, outr):
        s = jnp.sum(pr[...], axis=0)
        outr[...] = 1.0 - (1.0 - xr[...]) * (1.0 - jnp.exp(s))

    return pl.pallas_call(
        body,
        grid=(g,),
        in_specs=[pl.BlockSpec((_NW, blk), lambda i: (0, i)),
                  pl.BlockSpec((blk,), lambda i: (i,))],
        out_specs=pl.BlockSpec((blk,), lambda i: (i,)),
        out_shape=jax.ShapeDtypeStruct((n,), jnp.float32))(partials2, xc1)


def _tc_log(x1):
    def body(xr, outr):
        outr[...] = jnp.log(xr[...])
    return pl.pallas_call(
        body, out_shape=jax.ShapeDtypeStruct(x1.shape, jnp.float32))(x1)


def _tc_combine_b(partials2, xa1):
    c0 = float(_GAMMA) * math.log(_EPS)
    n = xa1.shape[0]
    g = 8
    blk = n // g

    def body(pr, xr, outr, lgr, lse_s, m_s):
        i = pl.program_id(0)
        s = jnp.sum(pr[...], axis=0)
        lse = _GAMMA * jnp.log(s + _EPS)
        lse_s[pl.ds(i * blk, blk)] = lse
        bm = jnp.max(lse)

        @pl.when(i == 0)
        def _():
            m_s[0] = bm

        @pl.when(i > 0)
        def _():
            m_s[0] = jnp.maximum(m_s[0], bm)

        @pl.when(i == g - 1)
        def _():
            m = jnp.maximum(m_s[0], c0)
            l_all = lse_s[...]
            l_all = jnp.where(m > 1.0, l_all / m, l_all)
            xn = 1.0 - (1.0 - xr[...]) * (1.0 - l_all)
            outr[...] = xn
            lgr[...] = jnp.log(xn)

    return pl.pallas_call(
        body,
        grid=(g,),
        in_specs=[pl.BlockSpec((_NW, blk), lambda i: (0, i)),
                  pl.BlockSpec((n,), lambda i: (0,))],
        out_specs=[pl.BlockSpec((n,), lambda i: (0,)),
                   pl.BlockSpec((n,), lambda i: (0,))],
        out_shape=[jax.ShapeDtypeStruct((n,), jnp.float32),
                   jax.ShapeDtypeStruct((n,), jnp.float32)],
        scratch_shapes=[pltpu.VMEM((n,), jnp.float32),
                        pltpu.SMEM((1,), jnp.float32)])(partials2, xa1)


def kernel(x, edge_index, clause_weights, edge_clause_index, edge_type,
           atom_node_idxs, conj_node_idxs, batch_size):
    eflat = edge_index.astype(jnp.int32)
    eci32 = edge_clause_index.astype(jnp.int32)
    wtbl = (clause_weights * (1.0 / _GAMMA)).astype(jnp.float32)

    x3 = x.reshape(_B, _PER)
    xa1 = x3[:, :_NA].reshape(-1).astype(jnp.float32)
    xc1 = x3[:, _NA:].reshape(-1).astype(jnp.float32)

    la1 = _tc_log(xa1)
    pa, pk_a = _pass_a2c_first(la1, eflat)
    xc1 = _tc_combine_a(pa, xc1)
    pb, pk_b, da = _pass_c2a_first(xc1, eflat, eci32, wtbl)
    xa1, la1 = _tc_combine_b(pb, xa1)
    for _ in range(_T - 1):
        pa = _pass_a2c(la1, pk_a)
        xc1 = _tc_combine_a(pa, xc1)
        pb = _pass_c2a(xc1, pk_b, da, wtbl)
        xa1, la1 = _tc_combine_b(pb, xa1)

    return xa1.reshape(_B, _NA)


# speedup vs baseline: 1.1003x; 1.1003x over previous
import functools
import math

import jax
import jax.numpy as jnp
from jax import lax
from jax.experimental import pallas as pl
from jax.experimental.pallas import tpu as pltpu
from jax.experimental.pallas import tpu_sc as plsc

_T = 5
_GAMMA = 0.015
_EPS = 0.0001

_B = 16
_NA = 2048
_NCJ = 1024
_PER = _NA + _NCJ
_NATOT = _B * _NA
_NCTOT = _B * _NCJ

_NCORES = 2
_NSUB = 16
_NW = _NCORES * _NSUB


def _mesh():
    return plsc.VectorSubcoreMesh(core_axis_name="c", subcore_axis_name="s")


def _zero_vmem(ref, n):
    @plsc.parallel_loop(0, n // 16, unroll=8)
    def _(i):
        ref[pl.ds(i * 16, 16)] = jnp.zeros((16,), jnp.float32)


def _bat16(n):
    return lax.shift_right_logical(lax.shift_right_logical(n, 10) * 21846, 16)


def _pass_a2c_first(la, eflat):
    e = eflat.shape[1] // 2
    ept = e // _NW

    @functools.partial(
        pl.kernel,
        out_type=[jax.ShapeDtypeStruct((_NW, _NCTOT), jnp.float32),
                  jax.ShapeDtypeStruct((e,), jnp.int32)],
        mesh=_mesh(),
        compiler_params=pltpu.CompilerParams(needs_layout_passes=False),
        scratch_types=[
            pltpu.VMEM((_NATOT,), jnp.float32),
            pltpu.VMEM((_NCTOT,), jnp.float32),
            pltpu.VMEM((ept,), jnp.int32),
            pltpu.VMEM((ept,), jnp.int32),
            pltpu.SemaphoreType.DMA,
            pltpu.SemaphoreType.DMA,
            pltpu.SemaphoreType.DMA,
        ],
    )
    def k(la_h, ef_h, out_h, pk_h, tbl, acc, sbuf, dbuf, sem, sem2, sem3):
        wid = lax.axis_index("s") * _NCORES + lax.axis_index("c")
        base = wid * ept
        d1 = pltpu.async_copy(la_h, tbl, sem)
        d2 = pltpu.async_copy(ef_h.at[0, pl.ds(base, ept)], sbuf, sem2)
        d3 = pltpu.async_copy(ef_h.at[1, pl.ds(base, ept)], dbuf, sem3)
        _zero_vmem(acc, _NCTOT)
        d2.wait()
        d3.wait()

        @plsc.parallel_loop(0, ept // 16, unroll=8)
        def _(j):
            srca = sbuf[pl.ds(j * 16, 16)]
            dstc = dbuf[pl.ds(j * 16, 16)]
            sa = srca - _bat16(srca) * _NCJ
            dc = dstc - _bat16(dstc) * _NA - _NA
            sbuf[pl.ds(j * 16, 16)] = sa | lax.shift_left(dc, 15)

        dout = pltpu.async_copy(sbuf, pk_h.at[pl.ds(base, ept)], sem2)
        d1.wait()

        @plsc.parallel_loop(0, ept // 16, unroll=16)
        def _(j):
            p = sbuf[pl.ds(j * 16, 16)]
            si = p & 0x7FFF
            di = lax.shift_right_logical(p, 15)
            v = plsc.load_gather(tbl, [si])
            plsc.addupdate_scatter(acc, [di], v)

        pltpu.sync_copy(acc, out_h.at[wid])
        dout.wait()

    return k(la, eflat)


def _pass_c2a_first(xc, eflat, eci, wtbl):
    e = eflat.shape[1] // 2
    ept = e // _NW
    ncl = wtbl.shape[0]

    @functools.partial(
        pl.kernel,
        out_type=[jax.ShapeDtypeStruct((_NW, _NATOT), jnp.float32),
                  jax.ShapeDtypeStruct((e,), jnp.int32),
                  jax.ShapeDtypeStruct((e,), jnp.int32)],
        mesh=_mesh(),
        compiler_params=pltpu.CompilerParams(needs_layout_passes=False),
        scratch_types=[
            pltpu.VMEM((_NCTOT,), jnp.float32),
            pltpu.VMEM((ncl,), jnp.float32),
            pltpu.VMEM((_NATOT,), jnp.float32),
            pltpu.VMEM((ept,), jnp.int32),
            pltpu.VMEM((ept,), jnp.int32),
            pltpu.VMEM((ept,), jnp.int32),
            pltpu.SemaphoreType.DMA,
            pltpu.SemaphoreType.DMA,
            pltpu.SemaphoreType.DMA,
            pltpu.SemaphoreType.DMA,
            pltpu.SemaphoreType.DMA,
        ],
    )
    def k(xc_h, ef_h, eci_h, w_h, out_h, pk_h, da_h,
          tbl, wv, acc, sbuf, dbuf, ebuf, sem, sem2, sem3, sem4, sem5):
        wid = lax.axis_index("s") * _NCORES + lax.axis_index("c")
        base = wid * ept
        d1 = pltpu.async_copy(xc_h, tbl, sem)
        d2 = pltpu.async_copy(w_h, wv, sem2)
        d3 = pltpu.async_copy(ef_h.at[0, pl.ds(e + base, ept)], sbuf, sem3)
        d4 = pltpu.async_copy(ef_h.at[1, pl.ds(e + base, ept)], dbuf, sem4)
        d5 = pltpu.async_copy(eci_h.at[pl.ds(e + base, ept)], ebuf, sem5)
        _zero_vmem(acc, _NATOT)
        d3.wait()
        d4.wait()
        d5.wait()

        @plsc.parallel_loop(0, ept // 16, unroll=8)
        def _(j):
            srcc = sbuf[pl.ds(j * 16, 16)]
            dsta = dbuf[pl.ds(j * 16, 16)]
            ec = ebuf[pl.ds(j * 16, 16)]
            sc = srcc - _bat16(srcc) * _NA - _NA
            sbuf[pl.ds(j * 16, 16)] = sc | lax.shift_left(ec, 14)
            dbuf[pl.ds(j * 16, 16)] = dsta - _bat16(dsta) * _NCJ

        do1 = pltpu.async_copy(sbuf, pk_h.at[pl.ds(base, ept)], sem3)
        do2 = pltpu.async_copy(dbuf, da_h.at[pl.ds(base, ept)], sem4)
        d1.wait()
        d2.wait()

        @plsc.parallel_loop(0, ept // 16, unroll=16)
        def _(j):
            p = sbuf[pl.ds(j * 16, 16)]
            di = dbuf[pl.ds(j * 16, 16)]
            si = p & 0x3FFF
            ki = lax.shift_right_logical(p, 14)
            v = plsc.load_gather(tbl, [si])
            w = plsc.load_gather(wv, [ki])
            plsc.addupdate_scatter(acc, [di], jnp.exp(v * w))

        pltpu.sync_copy(acc, out_h.at[wid])
        do1.wait()
        do2.wait()

    return k(xc, eflat, eci, wtbl)


def _pass_a2c(la, pk):
    e = pk.shape[0]
    ept = e // _NW

    @functools.partial(
        pl.kernel,
        out_type=jax.ShapeDtypeStruct((_NW, _NCTOT), jnp.float32),
        mesh=_mesh(),
        compiler_params=pltpu.CompilerParams(needs_layout_passes=False),
        scratch_types=[
            pltpu.VMEM((_NATOT,), jnp.float32),
            pltpu.VMEM((_NCTOT,), jnp.float32),
            pltpu.VMEM((ept,), jnp.int32),
            pltpu.SemaphoreType.DMA,
        ],
    )
    def k(la_h, pk_h, out_h, tbl, acc, pbuf, sem):
        wid = lax.axis_index("s") * _NCORES + lax.axis_index("c")
        base = wid * ept
        d1 = pltpu.async_copy(la_h, tbl, sem)
        d2 = pltpu.async_copy(pk_h.at[pl.ds(base, ept)], pbuf, sem)
        _zero_vmem(acc, _NCTOT)
        d1.wait()
        d2.wait()

        @plsc.parallel_loop(0, ept // 16, unroll=16)
        def _(j):
            p = pbuf[pl.ds(j * 16, 16)]
            si = p & 0x7FFF
            di = lax.shift_right_logical(p, 15)
            v = plsc.load_gather(tbl, [si])
            plsc.addupdate_scatter(acc, [di], v)

        pltpu.sync_copy(acc, out_h.at[wid])

    return k(la, pk)


def _pass_c2a(xc, pk, dst, wtbl):
    e = pk.shape[0]
    ept = e // _NW
    ncl = wtbl.shape[0]

    @functools.partial(
        pl.kernel,
        out_type=jax.ShapeDtypeStruct((_NW, _NATOT), jnp.float32),
        mesh=_mesh(),
        compiler_params=pltpu.CompilerParams(needs_layout_passes=False),
        scratch_types=[
            pltpu.VMEM((_NCTOT,), jnp.float32),
            pltpu.VMEM((ncl,), jnp.float32),
            pltpu.VMEM((_NATOT,), jnp.float32),
            pltpu.VMEM((ept,), jnp.int32),
            pltpu.VMEM((ept,), jnp.int32),
            pltpu.SemaphoreType.DMA,
        ],
    )
    def k(xc_h, pk_h, dst_h, w_h, out_h, tbl, wv, acc, pbuf, dbuf, sem):
        wid = lax.axis_index("s") * _NCORES + lax.axis_index("c")
        base = wid * ept
        d1 = pltpu.async_copy(xc_h, tbl, sem)
        d2 = pltpu.async_copy(w_h, wv, sem)
        d3 = pltpu.async_copy(pk_h.at[pl.ds(base, ept)], pbuf, sem)
        d4 = pltpu.async_copy(dst_h.at[pl.ds(base, ept)], dbuf, sem)
        _zero_vmem(acc, _NATOT)
        d1.wait()
        d2.wait()
        d3.wait()
        d4.wait()

        @plsc.parallel_loop(0, ept // 16, unroll=16)
        def _(j):
            p = pbuf[pl.ds(j * 16, 16)]
            di = dbuf[pl.ds(j * 16, 16)]
            si = p & 0x3FFF
            ki = lax.shift_right_logical(p, 14)
            v = plsc.load_gather(tbl, [si])
            w = plsc.load_gather(wv, [ki])
            plsc.addupdate_scatter(acc, [di], jnp.exp(v * w))

        pltpu.sync_copy(acc, out_h.at[wid])

    return k(xc, pk, dst, wtbl)


def _tc_combine_a(partials2, xc1):
    def body(pr, xr, outr):
        s = jnp.sum(pr[...], axis=0)
        agg = jnp.exp(s)
        outr[...] = 1.0 - (1.0 - xr[...]) * (1.0 - agg)
    return pl.pallas_call(
        body, out_shape=jax.ShapeDtypeStruct(xc1.shape, jnp.float32))(
            partials2, xc1)


def _tc_log(x1):
    def body(xr, outr):
        outr[...] = jnp.log(xr[...])
    return pl.pallas_call(
        body, out_shape=jax.ShapeDtypeStruct(x1.shape, jnp.float32))(x1)


def _tc_combine_b(partials2, xa1):
    c0 = float(_GAMMA) * math.log(_EPS)

    def body(pr, xr, outr, lgr):
        summed = jnp.sum(pr[...], axis=0)
        lse = _GAMMA * jnp.log(summed + _EPS)
        m = jnp.maximum(jnp.max(lse), c0)
        lse = jnp.where(m > 1.0, lse / m, lse)
        xn = 1.0 - (1.0 - xr[...]) * (1.0 - lse)
        outr[...] = xn
        lgr[...] = jnp.log(xn)

    return pl.pallas_call(
        body,
        out_shape=[jax.ShapeDtypeStruct(xa1.shape, jnp.float32),
                   jax.ShapeDtypeStruct(xa1.shape, jnp.float32)])(
            partials2, xa1)


def kernel(x, edge_index, clause_weights, edge_clause_index, edge_type,
           atom_node_idxs, conj_node_idxs, batch_size):
    eflat = edge_index.astype(jnp.int32)
    eci32 = edge_clause_index.astype(jnp.int32)
    wtbl = (clause_weights * (1.0 / _GAMMA)).astype(jnp.float32)

    x3 = x.reshape(_B, _PER)
    xa1 = x3[:, :_NA].reshape(-1).astype(jnp.float32)
    xc1 = x3[:, _NA:].reshape(-1).astype(jnp.float32)

    la1 = _tc_log(xa1)
    pa, pk_a = _pass_a2c_first(la1, eflat)
    xc1 = _tc_combine_a(pa, xc1)
    pb, pk_b, da = _pass_c2a_first(xc1, eflat, eci32, wtbl)
    xa1, la1 = _tc_combine_b(pb, xa1)
    for _ in range(_T - 1):
        pa = _pass_a2c(la1, pk_a)
        xc1 = _tc_combine_a(pa, xc1)
        pb = _pass_c2a(xc1, pk_b, da, wtbl)
        xa1, la1 = _tc_combine_b(pb, xa1)

    return xa1.reshape(_B, _NA)
